# baseline (speedup 1.0000x reference)
import functools

import jax
import jax.numpy as jnp
from jax import lax
from jax.experimental import pallas as pl
from jax.experimental.pallas import tpu as pltpu
from jax.experimental.pallas import tpu_sc as plsc

N = 10000
E = 320000
D = 128
DG = 64

NC = 2
NS = 16
NW = NC * NS

CHUNK = 128
CROWS = 2
BIG = CROWS * CHUNK
NBUF = 3
NCH_C = (66, 15)
NCH_MAX = max(NCH_C)

N_ACC = 10112
ROWS_ACC = N_ACC // NS

_sc_mesh = plsc.VectorSubcoreMesh(
    core_axis_name="c", subcore_axis_name="s", num_cores=NC)


@functools.partial(
    pl.kernel,
    out_type=jax.ShapeDtypeStruct((NC, N_ACC, DG), jnp.float32),
    mesh=_sc_mesh,
    scratch_types=[
        pltpu.VMEM((NCH_MAX, BIG), jnp.int32),
        pltpu.VMEM((NCH_MAX, BIG), jnp.int32),
        pltpu.VMEM((BIG, DG), jnp.float32),
        pltpu.VMEM((BIG, DG), jnp.float32),
        pltpu.VMEM((BIG, DG), jnp.float32),
        pltpu.VMEM_SHARED((N_ACC, DG), jnp.float32),
        pltpu.SemaphoreType.DMA,
        pltpu.SemaphoreType.DMA,
        pltpu.SemaphoreType.DMA,
    ],
    compiler_params=pltpu.CompilerParams(use_tc_tiling_on_sc=False,
                                         skip_device_barrier=True),
)
def _sc_segment_sum(h_hbm, src_hbm, dst_hbm, zero_hbm, out_hbm,
                    src_v, dst_v, buf0, buf1, buf2, acc_sh,
                    s0, s1, s2):
    bufs = (buf0, buf1, buf2)
    sems = (s0, s1, s2)
    cid = lax.axis_index("c")
    sid = lax.axis_index("s")
    wid = cid * NS + sid

    row0 = sid * ROWS_ACC
    with jax.named_scope("seg_init"):
        pltpu.sync_copy(zero_hbm.at[pl.ds(row0, ROWS_ACC)],
                        acc_sh.at[pl.ds(row0, ROWS_ACC)])

        pltpu.sync_copy(src_hbm.at[wid], src_v)
        pltpu.sync_copy(dst_hbm.at[wid], dst_v)
        plsc.subcore_barrier()

    def fire(j, rows_v, sem):
        pltpu.async_copy(h_hbm.at[src_v.at[j]], rows_v, sem)

    def gwait(j, rows_v, sem):
        pltpu.make_async_copy(h_hbm.at[src_v.at[j]], rows_v, sem).wait()

    def scatter(j, rows_v):
        pltpu.sync_copy(rows_v, acc_sh.at[dst_v.at[j]], add=True)

    n_my = lax.select(cid == 0, NCH_C[0], NCH_C[1])
    n_loop = lax.select(cid == 0, NCH_C[0] // NBUF - 1, NCH_C[1] // NBUF - 1)
    for b in range(NBUF):
        fire(b, bufs[b], sems[b])

    def body(g, carry):
        for b in range(NBUF):
            j = g * NBUF + b
            gwait(j, bufs[b], sems[b])
            scatter(j, bufs[b])
            fire(j + NBUF, bufs[b], sems[b])
        return carry

    with jax.named_scope("seg_edges"):
        lax.fori_loop(0, n_loop, body, 0)
        for b in range(NBUF):
            j = n_my - NBUF + b
            gwait(j, bufs[b], sems[b])
            scatter(j, bufs[b])
        plsc.subcore_barrier()

    with jax.named_scope("seg_out"):
        pltpu.sync_copy(acc_sh.at[pl.ds(row0, ROWS_ACC)],
                        out_hbm.at[cid, pl.ds(row0, ROWS_ACC)])


def _tc_mm_kernel(x_ref, w_ref, h_ref):
    h_ref[...] = jnp.maximum(
        jnp.dot(x_ref[...], w_ref[...], preferred_element_type=jnp.float32), 0.0)


def _tc_add_mm_kernel(x0_ref, p_ref, w_ref, y_ref, h_ref):
    y = x0_ref[...] + p_ref[0, :N] + p_ref[1, :N]
    y_ref[...] = y
    h_ref[...] = jnp.maximum(
        jnp.dot(y, w_ref[...], preferred_element_type=jnp.float32), 0.0)


def _tc_final_kernel(y0_ref, x1_ref, p_ref, out_ref):
    out_ref[:, :DG] = y0_ref[...]
    out_ref[:, DG:] = x1_ref[...] + p_ref[0, :N] + p_ref[1, :N]


_tc_mm = pl.pallas_call(
    _tc_mm_kernel,
    out_shape=jax.ShapeDtypeStruct((N, DG), jnp.float32),
)

_tc_add_mm = pl.pallas_call(
    _tc_add_mm_kernel,
    out_shape=(jax.ShapeDtypeStruct((N, DG), jnp.float32),
               jax.ShapeDtypeStruct((N, DG), jnp.float32)),
)

_tc_final = pl.pallas_call(
    _tc_final_kernel,
    out_shape=jax.ShapeDtypeStruct((N, D), jnp.float32),
)


@jax.jit
def kernel(x, edge_index, W0, W1):
    x0 = x[:, :DG]
    x1 = x[:, DG:]

    e0 = NS * NCH_C[0] * BIG
    e1 = NS * NCH_C[1] * BIG
    pad = e0 + e1 - E

    def _layout(a):
        parts = []
        for c, n_edges in ((0, e0), (1, e1)):
            lo = 0 if c == 0 else e0
            blk = a[lo:lo + n_edges].reshape(NS, NCH_C[c], BIG)
            blk = jnp.pad(blk, ((0, 0), (0, NCH_MAX - NCH_C[c]), (0, 0)))
            parts.append(blk)
        return jnp.concatenate(parts, axis=0)

    src = jnp.concatenate([edge_index[0], jnp.zeros((pad,), jnp.int32)])
    dst = jnp.concatenate([edge_index[1], jnp.full((pad,), N, jnp.int32)])
    src_r = _layout(src)
    dst_r = _layout(dst)
    zeros = jnp.zeros((N_ACC, DG), jnp.float32)

    h0 = _tc_mm(x1, W0)
    p0 = _sc_segment_sum(h0, src_r, dst_r, zeros)
    y0, h1 = _tc_add_mm(x0, p0, W1)
    p1 = _sc_segment_sum(h1, src_r, dst_r, zeros)
    return _tc_final(y0, x1, p1)


# speedup vs baseline: 1.0390x; 1.0390x over previous
import functools

import jax
import jax.numpy as jnp
from jax import lax
from jax.experimental import pallas as pl
from jax.experimental.pallas import tpu as pltpu
from jax.experimental.pallas import tpu_sc as plsc

N = 10000
E = 320000
D = 128
DG = 64

NC = 2
NS = 16
NW = NC * NS

CHUNK = 128
CROWS = 2
BIG = CROWS * CHUNK
NBUF = 3
NCH_C = (78, 3)
NCH_MAX = max(NCH_C)

N_ACC = 10112
ROWS_ACC = N_ACC // NS

_sc_mesh = plsc.VectorSubcoreMesh(
    core_axis_name="c", subcore_axis_name="s", num_cores=NC)


@functools.partial(
    pl.kernel,
    out_type=jax.ShapeDtypeStruct((NC, N_ACC, DG), jnp.float32),
    mesh=_sc_mesh,
    scratch_types=[
        pltpu.VMEM((NCH_MAX, BIG), jnp.int32),
        pltpu.VMEM((NCH_MAX, BIG), jnp.int32),
        pltpu.VMEM((BIG, DG), jnp.float32),
        pltpu.VMEM((BIG, DG), jnp.float32),
        pltpu.VMEM((BIG, DG), jnp.float32),
        pltpu.VMEM_SHARED((N_ACC, DG), jnp.float32),
        pltpu.SemaphoreType.DMA,
        pltpu.SemaphoreType.DMA,
        pltpu.SemaphoreType.DMA,
    ],
    compiler_params=pltpu.CompilerParams(use_tc_tiling_on_sc=False,
                                         skip_device_barrier=True),
)
def _sc_segment_sum(h_hbm, src_hbm, dst_hbm, zero_hbm, out_hbm,
                    src_v, dst_v, buf0, buf1, buf2, acc_sh,
                    s0, s1, s2):
    bufs = (buf0, buf1, buf2)
    sems = (s0, s1, s2)
    cid = lax.axis_index("c")
    sid = lax.axis_index("s")
    wid = cid * NS + sid

    row0 = sid * ROWS_ACC
    with jax.named_scope("seg_init"):
        pltpu.sync_copy(zero_hbm.at[pl.ds(row0, ROWS_ACC)],
                        acc_sh.at[pl.ds(row0, ROWS_ACC)])

        pltpu.sync_copy(src_hbm.at[wid], src_v)
        pltpu.sync_copy(dst_hbm.at[wid], dst_v)
        plsc.subcore_barrier()

    def fire(j, rows_v, sem):
        pltpu.async_copy(h_hbm.at[src_v.at[j]], rows_v, sem)

    def gwait(j, rows_v, sem):
        pltpu.make_async_copy(h_hbm.at[src_v.at[j]], rows_v, sem).wait()

    def scatter(j, rows_v):
        pltpu.sync_copy(rows_v, acc_sh.at[dst_v.at[j]], add=True)

    n_my = lax.select(cid == 0, NCH_C[0], NCH_C[1])
    n_loop = lax.select(cid == 0, NCH_C[0] // NBUF - 1, NCH_C[1] // NBUF - 1)
    for b in range(NBUF):
        fire(b, bufs[b], sems[b])

    def body(g, carry):
        for b in range(NBUF):
            j = g * NBUF + b
            gwait(j, bufs[b], sems[b])
            scatter(j, bufs[b])
            fire(j + NBUF, bufs[b], sems[b])
        return carry

    with jax.named_scope("seg_edges"):
        lax.fori_loop(0, n_loop, body, 0)
        for b in range(NBUF):
            j = n_my - NBUF + b
            gwait(j, bufs[b], sems[b])
            scatter(j, bufs[b])
        plsc.subcore_barrier()

    with jax.named_scope("seg_out"):
        pltpu.sync_copy(acc_sh.at[pl.ds(row0, ROWS_ACC)],
                        out_hbm.at[cid, pl.ds(row0, ROWS_ACC)])


def _tc_mm_kernel(x_ref, w_ref, h_ref):
    h_ref[...] = jnp.maximum(
        jnp.dot(x_ref[...], w_ref[...], preferred_element_type=jnp.float32), 0.0)


def _tc_add_mm_kernel(x0_ref, p_ref, w_ref, y_ref, h_ref):
    y = x0_ref[...] + p_ref[0, :N] + p_ref[1, :N]
    y_ref[...] = y
    h_ref[...] = jnp.maximum(
        jnp.dot(y, w_ref[...], preferred_element_type=jnp.float32), 0.0)


def _tc_final_kernel(y0_ref, x1_ref, p_ref, out_ref):
    out_ref[:, :DG] = y0_ref[...]
    out_ref[:, DG:] = x1_ref[...] + p_ref[0, :N] + p_ref[1, :N]


_tc_mm = pl.pallas_call(
    _tc_mm_kernel,
    out_shape=jax.ShapeDtypeStruct((N, DG), jnp.float32),
)

_tc_add_mm = pl.pallas_call(
    _tc_add_mm_kernel,
    out_shape=(jax.ShapeDtypeStruct((N, DG), jnp.float32),
               jax.ShapeDtypeStruct((N, DG), jnp.float32)),
)

_tc_final = pl.pallas_call(
    _tc_final_kernel,
    out_shape=jax.ShapeDtypeStruct((N, D), jnp.float32),
)


@jax.jit
def kernel(x, edge_index, W0, W1):
    x0 = x[:, :DG]
    x1 = x[:, DG:]

    e0 = NS * NCH_C[0] * BIG
    e1 = NS * NCH_C[1] * BIG
    pad = e0 + e1 - E

    def _layout(a):
        parts = []
        for c, n_edges in ((0, e0), (1, e1)):
            lo = 0 if c == 0 else e0
            blk = a[lo:lo + n_edges].reshape(NS, NCH_C[c], BIG)
            blk = jnp.pad(blk, ((0, 0), (0, NCH_MAX - NCH_C[c]), (0, 0)))
            parts.append(blk)
        return jnp.concatenate(parts, axis=0)

    src = jnp.concatenate([edge_index[0], jnp.zeros((pad,), jnp.int32)])
    dst = jnp.concatenate([edge_index[1], jnp.full((pad,), N, jnp.int32)])
    src_r = _layout(src)
    dst_r = _layout(dst)
    zeros = jnp.zeros((N_ACC, DG), jnp.float32)

    h0 = _tc_mm(x1, W0)
    p0 = _sc_segment_sum(h0, src_r, dst_r, zeros)
    y0, h1 = _tc_add_mm(x0, p0, W1)
    p1 = _sc_segment_sum(h1, src_r, dst_r, zeros)
    return _tc_final(y0, x1, p1)


# speedup vs baseline: 2.7841x; 2.6797x over previous
import functools

import jax
import jax.numpy as jnp
from jax import lax
from jax.experimental import pallas as pl
from jax.experimental.pallas import tpu as pltpu
from jax.experimental.pallas import tpu_sc as plsc

N = 10000
E = 320000
D = 128
DG = 64

NC = 2
NS = 16
NW = NC * NS

CHUNK = 128
CROWS = 1
BIG = CROWS * CHUNK
NBUF = 2
NCH_C = (80, 80)
NCH_MAX = max(NCH_C)

N_ACC = 10112
ROWS_ACC = N_ACC // NS
N_TAIL = N - (NS - 1) * ROWS_ACC

_sc_mesh = plsc.VectorSubcoreMesh(
    core_axis_name="c", subcore_axis_name="s", num_cores=NC)


@functools.partial(
    pl.kernel,
    out_type=jax.ShapeDtypeStruct((NC, N_ACC, DG), jnp.float32),
    mesh=_sc_mesh,
    scratch_types=[
        pltpu.VMEM((NCH_MAX, BIG), jnp.int32),
        pltpu.VMEM((NCH_MAX, BIG), jnp.int32),
        pltpu.VMEM((BIG, DG), jnp.float32),
        pltpu.VMEM((BIG, DG), jnp.float32),
        pltpu.VMEM_SHARED((N, DG), jnp.float32),
        pltpu.VMEM_SHARED((N_ACC, DG), jnp.float32),
        pltpu.SemaphoreType.DMA,
        pltpu.SemaphoreType.DMA,
    ],
    compiler_params=pltpu.CompilerParams(use_tc_tiling_on_sc=False,
                                         skip_device_barrier=True),
)
def _sc_segment_sum(h_hbm, src_hbm, dst_hbm, zero_hbm, out_hbm,
                    src_v, dst_v, buf0, buf1, h_sh, acc_sh, s0, s1):
    bufs = (buf0, buf1)
    sems = (s0, s1)
    cid = lax.axis_index("c")
    sid = lax.axis_index("s")
    wid = cid * NS + sid

    row0 = sid * ROWS_ACC
    with jax.named_scope("seg_init"):
        pltpu.sync_copy(zero_hbm.at[pl.ds(row0, ROWS_ACC)],
                        acc_sh.at[pl.ds(row0, ROWS_ACC)])

        @pl.when(sid < NS - 1)
        def _():
            pltpu.sync_copy(h_hbm.at[pl.ds(row0, ROWS_ACC)],
                            h_sh.at[pl.ds(row0, ROWS_ACC)])

        @pl.when(sid == NS - 1)
        def _():
            pltpu.sync_copy(h_hbm.at[pl.ds((NS - 1) * ROWS_ACC, N_TAIL)],
                            h_sh.at[pl.ds((NS - 1) * ROWS_ACC, N_TAIL)])

        pltpu.sync_copy(src_hbm.at[wid], src_v)
        pltpu.sync_copy(dst_hbm.at[wid], dst_v)
        plsc.subcore_barrier()

    def fire(j, rows_v, sem):
        pltpu.async_copy(h_sh.at[src_v.at[j]], rows_v, sem)

    def gwait(j, rows_v, sem):
        pltpu.make_async_copy(h_sh.at[src_v.at[j]], rows_v, sem).wait()

    def scatter(j, rows_v):
        pltpu.sync_copy(rows_v, acc_sh.at[dst_v.at[j]], add=True)

    n_my = lax.select(cid == 0, NCH_C[0], NCH_C[1])
    n_loop = lax.select(cid == 0, NCH_C[0] // NBUF - 1, NCH_C[1] // NBUF - 1)
    for b in range(NBUF):
        fire(b, bufs[b], sems[b])

    def body(g, carry):
        for b in range(NBUF):
            j = g * NBUF + b
            gwait(j, bufs[b], sems[b])
            scatter(j, bufs[b])
            fire(j + NBUF, bufs[b], sems[b])
        return carry

    with jax.named_scope("seg_edges"):
        lax.fori_loop(0, n_loop, body, 0)
        for b in range(NBUF):
            j = n_my - NBUF + b
            gwait(j, bufs[b], sems[b])
            scatter(j, bufs[b])
        plsc.subcore_barrier()

    with jax.named_scope("seg_out"):
        pltpu.sync_copy(acc_sh.at[pl.ds(row0, ROWS_ACC)],
                        out_hbm.at[cid, pl.ds(row0, ROWS_ACC)])


def _tc_mm_kernel(x_ref, w_ref, h_ref):
    h_ref[...] = jnp.maximum(
        jnp.dot(x_ref[...], w_ref[...], preferred_element_type=jnp.float32), 0.0)


def _tc_add_mm_kernel(x0_ref, p_ref, w_ref, y_ref, h_ref):
    y = x0_ref[...] + p_ref[0, :N] + p_ref[1, :N]
    y_ref[...] = y
    h_ref[...] = jnp.maximum(
        jnp.dot(y, w_ref[...], preferred_element_type=jnp.float32), 0.0)


def _tc_final_kernel(y0_ref, x1_ref, p_ref, out_ref):
    out_ref[:, :DG] = y0_ref[...]
    out_ref[:, DG:] = x1_ref[...] + p_ref[0, :N] + p_ref[1, :N]


_tc_mm = pl.pallas_call(
    _tc_mm_kernel,
    out_shape=jax.ShapeDtypeStruct((N, DG), jnp.float32),
)

_tc_add_mm = pl.pallas_call(
    _tc_add_mm_kernel,
    out_shape=(jax.ShapeDtypeStruct((N, DG), jnp.float32),
               jax.ShapeDtypeStruct((N, DG), jnp.float32)),
)

_tc_final = pl.pallas_call(
    _tc_final_kernel,
    out_shape=jax.ShapeDtypeStruct((N, D), jnp.float32),
)


@jax.jit
def kernel(x, edge_index, W0, W1):
    x0 = x[:, :DG]
    x1 = x[:, DG:]

    e0 = NS * NCH_C[0] * BIG
    e1 = NS * NCH_C[1] * BIG
    pad = e0 + e1 - E

    def _layout(a):
        parts = []
        for c, n_edges in ((0, e0), (1, e1)):
            lo = 0 if c == 0 else e0
            blk = a[lo:lo + n_edges].reshape(NS, NCH_C[c], BIG)
            blk = jnp.pad(blk, ((0, 0), (0, NCH_MAX - NCH_C[c]), (0, 0)))
            parts.append(blk)
        return jnp.concatenate(parts, axis=0)

    src = jnp.concatenate([edge_index[0], jnp.zeros((pad,), jnp.int32)])
    dst = jnp.concatenate([edge_index[1], jnp.full((pad,), N, jnp.int32)])
    src_r = _layout(src)
    dst_r = _layout(dst)
    zeros = jnp.zeros((N_ACC, DG), jnp.float32)

    h0 = _tc_mm(x1, W0)
    p0 = _sc_segment_sum(h0, src_r, dst_r, zeros)
    y0, h1 = _tc_add_mm(x0, p0, W1)
    p1 = _sc_segment_sum(h1, src_r, dst_r, zeros)
    return _tc_final(y0, x1, p1)
